# baseline (speedup 1.0000x reference)
import functools

import jax
import jax.numpy as jnp
from jax import lax
from jax.experimental import pallas as pl
from jax.experimental.pallas import tpu as pltpu
from jax.experimental.pallas import tpu_sc as plsc

D_MODEL = 1024
E = 64
D_FF = 128
N = 8192
CAP = 160
NSLOT = (E + 1) * CAP
NSLOT_PAD = 10752
ZERO_SLOT = E * CAP

_RB = 1024
_NB = N // _RB
_SIGMA = 0.01


def _k1_body(x_ref, wr_ref, nz_ref, gate_ref, eidx_ref, lb_ref, z_ref, x16_ref,
             p_acc, f_acc, z_acc):
    b = pl.program_id(0)
    xb = x_ref[...]
    x16_ref[...] = xb.astype(jnp.bfloat16)
    wr = wr_ref[...]
    nz = nz_ref[...]
    lc = lax.dot_general(xb, wr, (((1,), (1,)), ((), ())),
                         precision=lax.Precision.DEFAULT)

    mult = 1.0 + (nz * 2.0 - 1.0) * _SIGMA
    lg = lc * mult
    iota_e = lax.broadcasted_iota(jnp.int32, (_RB, E), 1)

    m = jnp.max(lg, axis=1, keepdims=True)
    eidx = jnp.min(jnp.where(lg == m, iota_e, E), axis=1, keepdims=True)
    s = jnp.sum(jnp.exp(lg - m), axis=1, keepdims=True)
    gate_ref[...] = 1.0 / s
    eidx_ref[...] = eidx

    mc = jnp.max(lc, axis=1, keepdims=True)
    eidx_c = jnp.min(jnp.where(lc == mc, iota_e, E), axis=1, keepdims=True)
    ec = jnp.exp(lc - mc)
    sc = jnp.sum(ec, axis=1, keepdims=True)
    probs_c = ec / sc
    z = mc + jnp.log(sc)

    @pl.when(b == 0)
    def _():
        p_acc[...] = jnp.zeros_like(p_acc)
        f_acc[...] = jnp.zeros_like(f_acc)
        z_acc[...] = jnp.zeros_like(z_acc)

    p_acc[...] += jnp.sum(probs_c, axis=0, keepdims=True)
    f_acc[...] += jnp.sum((iota_e == eidx_c).astype(jnp.float32), axis=0,
                          keepdims=True)
    z_acc[...] += jnp.sum(z * z).reshape(1, 1)

    @pl.when(b == _NB - 1)
    def _():
        p = p_acc[...] * (1.0 / N)
        f = f_acc[...] * (1.0 / N)
        lb_ref[...] = (E * jnp.sum(p * f)).reshape(1, 1)
        z_ref[...] = z_acc[...] * (1.0 / N)


_k1 = pl.pallas_call(
    _k1_body,
    grid=(_NB,),
    in_specs=[
        pl.BlockSpec((_RB, D_MODEL), lambda b: (b, 0)),
        pl.BlockSpec((E, D_MODEL), lambda b: (0, 0)),
        pl.BlockSpec((_RB, E), lambda b: (b, 0)),
    ],
    out_specs=[
        pl.BlockSpec((_RB, 1), lambda b: (b, 0)),
        pl.BlockSpec((_RB, 1), lambda b: (b, 0)),
        pl.BlockSpec((1, 1), lambda b: (0, 0)),
        pl.BlockSpec((1, 1), lambda b: (0, 0)),
        pl.BlockSpec((_RB, D_MODEL), lambda b: (b, 0)),
    ],
    out_shape=[
        jax.ShapeDtypeStruct((N, 1), jnp.float32),
        jax.ShapeDtypeStruct((N, 1), jnp.int32),
        jax.ShapeDtypeStruct((1, 1), jnp.float32),
        jax.ShapeDtypeStruct((1, 1), jnp.float32),
        jax.ShapeDtypeStruct((N, D_MODEL), jnp.bfloat16),
    ],
    scratch_shapes=[
        pltpu.VMEM((1, E), jnp.float32),
        pltpu.VMEM((1, E), jnp.float32),
        pltpu.VMEM((1, 1), jnp.float32),
    ],
)


_CHUNK = 512
_NCH = N // _CHUNK


def _k2_body(gate_ref, eidx_ref, src_ref, keep_ref):
    g = gate_ref[...]
    e = eidx_ref[...]
    gb = lax.bitcast_convert_type(g, jnp.int32)
    iota_e = lax.broadcasted_iota(jnp.int32, (N, E), 1)
    aexp = (e == iota_e)
    af = aexp.astype(jnp.float32)
    tok = lax.broadcasted_iota(jnp.int32, (N, 1), 0)
    key2 = (N - 1) - tok

    cnt = jnp.sum(af, axis=0, keepdims=True)
    cnt_tok = jnp.sum(af * cnt, axis=1, keepdims=True)
    keep_all = cnt_tok <= CAP

    digits = [
        jnp.bitwise_and(lax.shift_right_logical(gb, 24), 255),
        jnp.bitwise_and(lax.shift_right_logical(gb, 16), 255),
        jnp.bitwise_and(lax.shift_right_logical(gb, 8), 255),
        jnp.bitwise_and(gb, 255),
        lax.shift_right_logical(key2, 7),
        jnp.bitwise_and(key2, 127),
    ]

    nbuck = 256
    iota_b_tok = lax.broadcasted_iota(jnp.int32, (N, nbuck), 1)
    iota_b_e = lax.broadcasted_iota(jnp.int32, (E, nbuck), 1)
    r2d = lax.broadcasted_iota(jnp.int32, (nbuck, nbuck), 0)
    c2d = lax.broadcasted_iota(jnp.int32, (nbuck, nbuck), 1)
    tri_ge = (r2d >= c2d).astype(jnp.float32)

    active = jnp.ones((N, 1), dtype=jnp.bool_)
    rem = jnp.full((E, 1), float(CAP), jnp.float32)
    gt_acc = jnp.zeros((N, 1), dtype=jnp.bool_)
    eq_acc = jnp.ones((N, 1), dtype=jnp.bool_)

    for d in digits:
        d1h = (d == iota_b_tok).astype(jnp.float32)
        am = af * active.astype(jnp.float32)
        hist = lax.dot_general(am, d1h, (((0,), (0,)), ((), ())),
                               precision=lax.Precision.DEFAULT)
        rcsum = lax.dot_general(hist, tri_ge, (((1,), (0,)), ((), ())),
                                precision=lax.Precision.HIGHEST)
        ok = rcsum >= rem
        c_e = jnp.max(jnp.where(ok, iota_b_e, 0), axis=1, keepdims=True)
        c1h = (iota_b_e == c_e).astype(jnp.float32)
        h_at = jnp.sum(hist * c1h, axis=1, keepdims=True)
        rc_at = jnp.sum(rcsum * c1h, axis=1, keepdims=True)
        rem = rem - (rc_at - h_at)
        ce_tok = lax.dot_general(af, c_e.astype(jnp.float32),
                                 (((1,), (0,)), ((), ()))).astype(jnp.int32)
        gt_p = d > ce_tok
        eq_p = d == ce_tok
        gt_acc = gt_acc | (eq_acc & gt_p)
        eq_acc = eq_acc & eq_p
        active = active & eq_p

    keep = keep_all | gt_acc | eq_acc
    keep_ref[...] = keep.astype(jnp.int32)

    kf = keep.astype(jnp.float32) * af
    rr = lax.broadcasted_iota(jnp.int32, (_CHUNK, _CHUNK), 0)
    cc = lax.broadcasted_iota(jnp.int32, (_CHUNK, _CHUNK), 1)
    tri_lt = (cc < rr).astype(jnp.float32)
    carry = jnp.zeros((1, E), jnp.float32)
    for c in range(_NCH):
        lo, hi = c * _CHUNK, (c + 1) * _CHUNK
        blk = kf[lo:hi, :]
        excl = lax.dot_general(tri_lt, blk, (((1,), (0,)), ((), ())),
                               precision=lax.Precision.DEFAULT)
        slot_e = excl + carry
        carry = carry + jnp.sum(blk, axis=0, keepdims=True)
        slot = jnp.sum(slot_e * af[lo:hi, :], axis=1,
                       keepdims=True).astype(jnp.int32)
        srcv = jnp.where(keep[lo:hi, :], e[lo:hi, :] * CAP + slot, ZERO_SLOT)
        src_ref[lo:hi, :] = srcv


_k2 = pl.pallas_call(
    _k2_body,
    in_specs=[
        pl.BlockSpec((N, 1), lambda: (0, 0)),
        pl.BlockSpec((N, 1), lambda: (0, 0)),
    ],
    out_specs=[
        pl.BlockSpec((N, 1), lambda: (0, 0)),
        pl.BlockSpec((N, 1), lambda: (0, 0)),
    ],
    out_shape=[
        jax.ShapeDtypeStruct((N, 1), jnp.int32),
        jax.ShapeDtypeStruct((N, 1), jnp.int32),
    ],
)


def _k2b_body(src_hbm, keep_hbm, gate_hbm, topi_hbm, gslot_hbm,
              src_v, keep_v, gate_v, topi_v, gslot_v):
    wid = lax.axis_index("s") * 2 + lax.axis_index("c")

    @pl.when(wid == 0)
    def _():
        pltpu.sync_copy(src_hbm, src_v)
        pltpu.sync_copy(keep_hbm, keep_v)
        pltpu.sync_copy(gate_hbm, gate_v)

        def zero_topi(i, _):
            topi_v[pl.ds(i * 16, 16)] = jnp.zeros((16,), jnp.int32)
            return 0
        lax.fori_loop(0, NSLOT_PAD // 16, zero_topi, 0)

        def zero_g(i, _):
            gslot_v[pl.ds(i * 16, 16)] = jnp.zeros((16,), jnp.float32)
            return 0
        lax.fori_loop(0, NSLOT // 16, zero_g, 0)

        def scat(j, _):
            sl = src_v[pl.ds(j * 16, 16)]
            kp = keep_v[pl.ds(j * 16, 16)] != 0
            gv = gate_v[pl.ds(j * 16, 16)]
            ids = lax.iota(jnp.int32, 16) + j * 16
            plsc.store_scatter(topi_v, [sl], ids, mask=kp)
            plsc.store_scatter(gslot_v, [sl], gv, mask=kp)
            return 0
        lax.fori_loop(0, N // 16, scat, 0)

        pltpu.sync_copy(topi_v, topi_hbm)
        pltpu.sync_copy(gslot_v, gslot_hbm)


_K3_PW = NSLOT_PAD // 32
_K3_CH = 56


def _gather_rows_2buf(src_hbm, idx_v, out_hbm, base, nch, ch, bufs, gsems, wsems):
    writes = [None, None]
    cp_g = pltpu.async_copy(src_hbm.at[idx_v.at[pl.ds(0, ch)]], bufs[0],
                            gsems[0])
    for c in range(nch):
        b = c % 2
        nxt = None
        if c + 1 < nch:
            if writes[1 - b] is not None:
                writes[1 - b].wait()
                writes[1 - b] = None
            nxt = pltpu.async_copy(
                src_hbm.at[idx_v.at[pl.ds((c + 1) * ch, ch)]],
                bufs[1 - b], gsems[1 - b])
        cp_g.wait()
        writes[b] = pltpu.async_copy(
            bufs[b], out_hbm.at[pl.ds(base + c * ch, ch)], wsems[b])
        cp_g = nxt
    for b in range(2):
        if writes[b] is not None:
            writes[b].wait()


def _k3_body(topi_hbm, x_hbm, xg_hbm, idx_v, buf0, buf1, gs0, gs1, ws0, ws1):
    wid = lax.axis_index("s") * 2 + lax.axis_index("c")
    base = wid * _K3_PW
    pltpu.sync_copy(topi_hbm.at[pl.ds(base, _K3_PW)], idx_v)
    _gather_rows_2buf(x_hbm, idx_v, xg_hbm, base, _K3_PW // _K3_CH, _K3_CH,
                      (buf0, buf1), (gs0, gs1), (ws0, ws1))


def _k4_body(xg_ref, w1_ref, w2_ref, g_ref, out_ref):
    xb = xg_ref[...].astype(jnp.float32)
    h = jnp.maximum(
        lax.dot_general(xb, w1_ref[0], (((1,), (0,)), ((), ())),
                        precision=lax.Precision.DEFAULT), 0.0)
    y = lax.dot_general(h, w2_ref[0], (((1,), (0,)), ((), ())),
                        precision=lax.Precision.DEFAULT)
    out_ref[0] = y * g_ref[...]


_k4 = pl.pallas_call(
    _k4_body,
    grid=(E + 1,),
    in_specs=[
        pl.BlockSpec((CAP, D_MODEL), lambda e: (e, 0)),
        pl.BlockSpec((1, D_MODEL, D_FF), lambda e: (jnp.minimum(e, E - 1), 0, 0)),
        pl.BlockSpec((1, D_FF, D_MODEL), lambda e: (jnp.minimum(e, E - 1), 0, 0)),
        pl.BlockSpec((CAP, 1), lambda e: (e, 0)),
    ],
    out_specs=pl.BlockSpec((1, CAP, D_MODEL), lambda e: (e, 0, 0)),
    out_shape=jax.ShapeDtypeStruct((E + 1, CAP, D_MODEL), jnp.float32),
)


_K5_PW = N // 32
_K5_CH = 32


def _k5_body(src_hbm, ybuf_hbm, y_hbm, idx_v, buf0, buf1, gs0, gs1, ws0, ws1):
    wid = lax.axis_index("s") * 2 + lax.axis_index("c")
    base = wid * _K5_PW
    pltpu.sync_copy(src_hbm.at[pl.ds(base, _K5_PW)], idx_v)
    _gather_rows_2buf(ybuf_hbm, idx_v, y_hbm, base, _K5_PW // _K5_CH, _K5_CH,
                      (buf0, buf1), (gs0, gs1), (ws0, ws1))


@functools.lru_cache(maxsize=1)
def _sc_kernels():
    mesh = plsc.VectorSubcoreMesh(core_axis_name="c", subcore_axis_name="s")
    cp = pltpu.CompilerParams(needs_layout_passes=False)
    k2b = pl.kernel(
        _k2b_body,
        mesh=mesh,
        compiler_params=cp,
        out_type=[
            jax.ShapeDtypeStruct((NSLOT_PAD,), jnp.int32),
            jax.ShapeDtypeStruct((NSLOT,), jnp.float32),
        ],
        scratch_types=[
            pltpu.VMEM((N,), jnp.int32),
            pltpu.VMEM((N,), jnp.int32),
            pltpu.VMEM((N,), jnp.float32),
            pltpu.VMEM((NSLOT_PAD,), jnp.int32),
            pltpu.VMEM((NSLOT,), jnp.float32),
        ],
    )
    k3 = pl.kernel(
        _k3_body,
        mesh=mesh,
        out_type=jax.ShapeDtypeStruct((NSLOT_PAD, D_MODEL // 2), jnp.int32),
        scratch_types=[
            pltpu.VMEM((_K3_PW,), jnp.int32),
            pltpu.VMEM((_K3_CH, D_MODEL // 2), jnp.int32),
            pltpu.VMEM((_K3_CH, D_MODEL // 2), jnp.int32),
            pltpu.SemaphoreType.DMA,
            pltpu.SemaphoreType.DMA,
            pltpu.SemaphoreType.DMA,
            pltpu.SemaphoreType.DMA,
        ],
    )
    k5 = pl.kernel(
        _k5_body,
        mesh=mesh,
        out_type=jax.ShapeDtypeStruct((N, D_MODEL), jnp.float32),
        scratch_types=[
            pltpu.VMEM((_K5_PW,), jnp.int32),
            pltpu.VMEM((_K5_CH, D_MODEL), jnp.float32),
            pltpu.VMEM((_K5_CH, D_MODEL), jnp.float32),
            pltpu.SemaphoreType.DMA,
            pltpu.SemaphoreType.DMA,
            pltpu.SemaphoreType.DMA,
            pltpu.SemaphoreType.DMA,
        ],
    )
    return k2b, k3, k5


def kernel(x, W_router, W1, W2, noise, token_mask):
    k2b, k3, k5 = _sc_kernels()
    xf = x.reshape(N, D_MODEL)
    gate2, eidx2, lb11, z11, x16 = _k1(xf, W_router, noise)
    src2, keep2 = _k2(gate2, eidx2)
    src1 = src2.reshape(N)
    topi, gslot = k2b(src1, keep2.reshape(N), gate2.reshape(N))
    x16_i32 = lax.bitcast_convert_type(
        x16.reshape(N, D_MODEL // 2, 2), jnp.int32)
    xg_i32 = k3(topi, x16_i32)
    xg16 = lax.bitcast_convert_type(
        xg_i32, jnp.bfloat16).reshape(NSLOT_PAD, D_MODEL)
    ybuf = _k4(xg16, W1, W2, gslot.reshape(NSLOT, 1))
    y = k5(src1, ybuf.reshape(NSLOT, D_MODEL))
    return (y.reshape(x.shape), lb11[0, 0], z11[0, 0])


# speedup vs baseline: 1.8393x; 1.8393x over previous
import functools

import jax
import jax.numpy as jnp
from jax import lax
from jax.experimental import pallas as pl
from jax.experimental.pallas import tpu as pltpu
from jax.experimental.pallas import tpu_sc as plsc

D_MODEL = 1024
E = 64
D_FF = 128
N = 8192
CAP = 160
NSLOT = (E + 1) * CAP
NSLOT_PAD = 10752
ZERO_SLOT = E * CAP

_RB = 1024
_NB = N // _RB
_SIGMA = 0.01


def _k1_body(x_ref, wr_ref, nz_ref, gate_ref, eidx_ref, lb_ref, z_ref,
             p_acc, f_acc, z_acc):
    b = pl.program_id(0)
    xb = x_ref[...]
    wr = wr_ref[...]
    nz = nz_ref[...]
    lc = lax.dot_general(xb, wr, (((1,), (1,)), ((), ())),
                         precision=lax.Precision.DEFAULT)

    mult = 1.0 + (nz * 2.0 - 1.0) * _SIGMA
    lg = lc * mult
    iota_e = lax.broadcasted_iota(jnp.int32, (_RB, E), 1)

    m = jnp.max(lg, axis=1, keepdims=True)
    eidx = jnp.min(jnp.where(lg == m, iota_e, E), axis=1, keepdims=True)
    s = jnp.sum(jnp.exp(lg - m), axis=1, keepdims=True)
    gate_ref[...] = 1.0 / s
    eidx_ref[...] = eidx

    mc = jnp.max(lc, axis=1, keepdims=True)
    eidx_c = jnp.min(jnp.where(lc == mc, iota_e, E), axis=1, keepdims=True)
    ec = jnp.exp(lc - mc)
    sc = jnp.sum(ec, axis=1, keepdims=True)
    probs_c = ec / sc
    z = mc + jnp.log(sc)

    @pl.when(b == 0)
    def _():
        p_acc[...] = jnp.zeros_like(p_acc)
        f_acc[...] = jnp.zeros_like(f_acc)
        z_acc[...] = jnp.zeros_like(z_acc)

    p_acc[...] += jnp.sum(probs_c, axis=0, keepdims=True)
    f_acc[...] += jnp.sum((iota_e == eidx_c).astype(jnp.float32), axis=0,
                          keepdims=True)
    z_acc[...] += jnp.sum(z * z).reshape(1, 1)

    @pl.when(b == _NB - 1)
    def _():
        p = p_acc[...] * (1.0 / N)
        f = f_acc[...] * (1.0 / N)
        lb_ref[...] = (E * jnp.sum(p * f)).reshape(1, 1)
        z_ref[...] = z_acc[...] * (1.0 / N)


_k1 = pl.pallas_call(
    _k1_body,
    grid=(_NB,),
    in_specs=[
        pl.BlockSpec((_RB, D_MODEL), lambda b: (b, 0)),
        pl.BlockSpec((E, D_MODEL), lambda b: (0, 0)),
        pl.BlockSpec((_RB, E), lambda b: (b, 0)),
    ],
    out_specs=[
        pl.BlockSpec((_RB, 1), lambda b: (b, 0)),
        pl.BlockSpec((_RB, 1), lambda b: (b, 0)),
        pl.BlockSpec((1, 1), lambda b: (0, 0)),
        pl.BlockSpec((1, 1), lambda b: (0, 0)),
    ],
    out_shape=[
        jax.ShapeDtypeStruct((N, 1), jnp.float32),
        jax.ShapeDtypeStruct((N, 1), jnp.int32),
        jax.ShapeDtypeStruct((1, 1), jnp.float32),
        jax.ShapeDtypeStruct((1, 1), jnp.float32),
    ],
    scratch_shapes=[
        pltpu.VMEM((1, E), jnp.float32),
        pltpu.VMEM((1, E), jnp.float32),
        pltpu.VMEM((1, 1), jnp.float32),
    ],
)


_CHUNK = 512
_NCH = N // _CHUNK


def _k2_body(gate_ref, eidx_ref, src_ref, keep_ref):
    g = gate_ref[...]
    e = eidx_ref[...]
    gb = lax.bitcast_convert_type(g, jnp.int32)
    iota_e = lax.broadcasted_iota(jnp.int32, (N, E), 1)
    aexp = (e == iota_e)
    af = aexp.astype(jnp.float32)
    tok = lax.broadcasted_iota(jnp.int32, (N, 1), 0)
    key2 = (N - 1) - tok

    cnt = jnp.sum(af, axis=0, keepdims=True)
    cnt_tok = jnp.sum(af * cnt, axis=1, keepdims=True)
    keep_all = cnt_tok <= CAP

    digits = [
        jnp.bitwise_and(lax.shift_right_logical(gb, 24), 255),
        jnp.bitwise_and(lax.shift_right_logical(gb, 16), 255),
        jnp.bitwise_and(lax.shift_right_logical(gb, 8), 255),
        jnp.bitwise_and(gb, 255),
        lax.shift_right_logical(key2, 7),
        jnp.bitwise_and(key2, 127),
    ]

    nbuck = 256
    iota_b_tok = lax.broadcasted_iota(jnp.int32, (N, nbuck), 1)
    iota_b_e = lax.broadcasted_iota(jnp.int32, (E, nbuck), 1)
    r2d = lax.broadcasted_iota(jnp.int32, (nbuck, nbuck), 0)
    c2d = lax.broadcasted_iota(jnp.int32, (nbuck, nbuck), 1)
    tri_ge = (r2d >= c2d).astype(jnp.float32)

    active = jnp.ones((N, 1), dtype=jnp.bool_)
    rem = jnp.full((E, 1), float(CAP), jnp.float32)
    gt_acc = jnp.zeros((N, 1), dtype=jnp.bool_)
    eq_acc = jnp.ones((N, 1), dtype=jnp.bool_)

    for d in digits:
        d1h = (d == iota_b_tok).astype(jnp.float32)
        am = af * active.astype(jnp.float32)
        hist = lax.dot_general(am, d1h, (((0,), (0,)), ((), ())),
                               precision=lax.Precision.DEFAULT)
        rcsum = lax.dot_general(hist, tri_ge, (((1,), (0,)), ((), ())),
                                precision=lax.Precision.HIGHEST)
        ok = rcsum >= rem
        c_e = jnp.max(jnp.where(ok, iota_b_e, 0), axis=1, keepdims=True)
        c1h = (iota_b_e == c_e).astype(jnp.float32)
        h_at = jnp.sum(hist * c1h, axis=1, keepdims=True)
        rc_at = jnp.sum(rcsum * c1h, axis=1, keepdims=True)
        rem = rem - (rc_at - h_at)
        ce_tok = lax.dot_general(af, c_e.astype(jnp.float32),
                                 (((1,), (0,)), ((), ()))).astype(jnp.int32)
        gt_p = d > ce_tok
        eq_p = d == ce_tok
        gt_acc = gt_acc | (eq_acc & gt_p)
        eq_acc = eq_acc & eq_p
        active = active & eq_p

    keep = keep_all | gt_acc | eq_acc
    keep_ref[...] = keep.astype(jnp.int32)

    kf = keep.astype(jnp.float32) * af
    rr = lax.broadcasted_iota(jnp.int32, (_CHUNK, _CHUNK), 0)
    cc = lax.broadcasted_iota(jnp.int32, (_CHUNK, _CHUNK), 1)
    tri_lt = (cc < rr).astype(jnp.float32)
    carry = jnp.zeros((1, E), jnp.float32)
    for c in range(_NCH):
        lo, hi = c * _CHUNK, (c + 1) * _CHUNK
        blk = kf[lo:hi, :]
        excl = lax.dot_general(tri_lt, blk, (((1,), (0,)), ((), ())),
                               precision=lax.Precision.DEFAULT)
        slot_e = excl + carry
        carry = carry + jnp.sum(blk, axis=0, keepdims=True)
        slot = jnp.sum(slot_e * af[lo:hi, :], axis=1,
                       keepdims=True).astype(jnp.int32)
        srcv = jnp.where(keep[lo:hi, :], e[lo:hi, :] * CAP + slot, ZERO_SLOT)
        src_ref[lo:hi, :] = srcv


_k2 = pl.pallas_call(
    _k2_body,
    in_specs=[
        pl.BlockSpec((N, 1), lambda: (0, 0)),
        pl.BlockSpec((N, 1), lambda: (0, 0)),
    ],
    out_specs=[
        pl.BlockSpec((N, 1), lambda: (0, 0)),
        pl.BlockSpec((N, 1), lambda: (0, 0)),
    ],
    out_shape=[
        jax.ShapeDtypeStruct((N, 1), jnp.int32),
        jax.ShapeDtypeStruct((N, 1), jnp.int32),
    ],
)


def _k2b_body(src_hbm, keep_hbm, gate_hbm, topi_hbm, gslot_hbm,
              src_v, keep_v, gate_v, topi_v, gslot_v):
    wid = lax.axis_index("s") * 2 + lax.axis_index("c")

    @pl.when(wid == 0)
    def _():
        pltpu.sync_copy(src_hbm, src_v)
        pltpu.sync_copy(keep_hbm, keep_v)
        pltpu.sync_copy(gate_hbm, gate_v)

        def zero_topi(i, _):
            topi_v[pl.ds(i * 16, 16)] = jnp.zeros((16,), jnp.int32)
            return 0
        lax.fori_loop(0, NSLOT_PAD // 16, zero_topi, 0)

        def zero_g(i, _):
            gslot_v[pl.ds(i * 16, 16)] = jnp.zeros((16,), jnp.float32)
            return 0
        lax.fori_loop(0, NSLOT // 16, zero_g, 0)

        def scat(j, _):
            sl = src_v[pl.ds(j * 16, 16)]
            kp = keep_v[pl.ds(j * 16, 16)] != 0
            gv = gate_v[pl.ds(j * 16, 16)]
            ids = lax.iota(jnp.int32, 16) + j * 16
            plsc.store_scatter(topi_v, [sl], ids, mask=kp)
            plsc.store_scatter(gslot_v, [sl], gv, mask=kp)
            return 0
        lax.fori_loop(0, N // 16, scat, 0)

        pltpu.sync_copy(topi_v, topi_hbm)
        pltpu.sync_copy(gslot_v, gslot_hbm)


_K3_PW = NSLOT_PAD // 32
_K3_CH = 56


def _gather_rows_2buf(src_hbm, idx_v, out_hbm, base, nch, ch, bufs, gsems, wsems):
    writes = [None, None]
    cp_g = pltpu.async_copy(src_hbm.at[idx_v.at[pl.ds(0, ch)]], bufs[0],
                            gsems[0])
    for c in range(nch):
        b = c % 2
        nxt = None
        if c + 1 < nch:
            if writes[1 - b] is not None:
                writes[1 - b].wait()
                writes[1 - b] = None
            nxt = pltpu.async_copy(
                src_hbm.at[idx_v.at[pl.ds((c + 1) * ch, ch)]],
                bufs[1 - b], gsems[1 - b])
        cp_g.wait()
        writes[b] = pltpu.async_copy(
            bufs[b], out_hbm.at[pl.ds(base + c * ch, ch)], wsems[b])
        cp_g = nxt
    for b in range(2):
        if writes[b] is not None:
            writes[b].wait()


def _k3_body(topi_hbm, x_hbm, xg_hbm, idx_v, buf0, buf1, gs0, gs1, ws0, ws1):
    wid = lax.axis_index("s") * 2 + lax.axis_index("c")
    base = wid * _K3_PW
    pltpu.sync_copy(topi_hbm.at[pl.ds(base, _K3_PW)], idx_v)
    _gather_rows_2buf(x_hbm, idx_v, xg_hbm, base, _K3_PW // _K3_CH, _K3_CH,
                      (buf0, buf1), (gs0, gs1), (ws0, ws1))


def _k4_body(xg_ref, w1_ref, w2_ref, g_ref, out_ref):
    xb = xg_ref[...]
    h = jnp.maximum(
        lax.dot_general(xb, w1_ref[0], (((1,), (0,)), ((), ())),
                        precision=lax.Precision.DEFAULT), 0.0)
    y = lax.dot_general(h, w2_ref[0], (((1,), (0,)), ((), ())),
                        precision=lax.Precision.DEFAULT)
    out_ref[0] = y * g_ref[...]


_k4 = pl.pallas_call(
    _k4_body,
    grid=(E + 1,),
    in_specs=[
        pl.BlockSpec((CAP, D_MODEL), lambda e: (e, 0)),
        pl.BlockSpec((1, D_MODEL, D_FF), lambda e: (jnp.minimum(e, E - 1), 0, 0)),
        pl.BlockSpec((1, D_FF, D_MODEL), lambda e: (jnp.minimum(e, E - 1), 0, 0)),
        pl.BlockSpec((CAP, 1), lambda e: (e, 0)),
    ],
    out_specs=pl.BlockSpec((1, CAP, D_MODEL), lambda e: (e, 0, 0)),
    out_shape=jax.ShapeDtypeStruct((E + 1, CAP, D_MODEL), jnp.float32),
)


_K5_PW = N // 32
_K5_CH = 32


def _k5_body(src_hbm, ybuf_hbm, y_hbm, idx_v, buf0, buf1, gs0, gs1, ws0, ws1):
    wid = lax.axis_index("s") * 2 + lax.axis_index("c")
    base = wid * _K5_PW
    pltpu.sync_copy(src_hbm.at[pl.ds(base, _K5_PW)], idx_v)
    _gather_rows_2buf(ybuf_hbm, idx_v, y_hbm, base, _K5_PW // _K5_CH, _K5_CH,
                      (buf0, buf1), (gs0, gs1), (ws0, ws1))


@functools.lru_cache(maxsize=1)
def _sc_kernels():
    mesh = plsc.VectorSubcoreMesh(core_axis_name="c", subcore_axis_name="s")
    cp = pltpu.CompilerParams(needs_layout_passes=False)
    k2b = pl.kernel(
        _k2b_body,
        mesh=mesh,
        compiler_params=cp,
        out_type=[
            jax.ShapeDtypeStruct((NSLOT_PAD,), jnp.int32),
            jax.ShapeDtypeStruct((NSLOT,), jnp.float32),
        ],
        scratch_types=[
            pltpu.VMEM((N,), jnp.int32),
            pltpu.VMEM((N,), jnp.int32),
            pltpu.VMEM((N,), jnp.float32),
            pltpu.VMEM((NSLOT_PAD,), jnp.int32),
            pltpu.VMEM((NSLOT,), jnp.float32),
        ],
    )
    k3 = pl.kernel(
        _k3_body,
        mesh=mesh,
        out_type=jax.ShapeDtypeStruct((NSLOT_PAD, D_MODEL), jnp.float32),
        scratch_types=[
            pltpu.VMEM((_K3_PW,), jnp.int32),
            pltpu.VMEM((_K3_CH, D_MODEL), jnp.float32),
            pltpu.VMEM((_K3_CH, D_MODEL), jnp.float32),
            pltpu.SemaphoreType.DMA,
            pltpu.SemaphoreType.DMA,
            pltpu.SemaphoreType.DMA,
            pltpu.SemaphoreType.DMA,
        ],
    )
    k5 = pl.kernel(
        _k5_body,
        mesh=mesh,
        out_type=jax.ShapeDtypeStruct((N, D_MODEL), jnp.float32),
        scratch_types=[
            pltpu.VMEM((_K5_PW,), jnp.int32),
            pltpu.VMEM((_K5_CH, D_MODEL), jnp.float32),
            pltpu.VMEM((_K5_CH, D_MODEL), jnp.float32),
            pltpu.SemaphoreType.DMA,
            pltpu.SemaphoreType.DMA,
            pltpu.SemaphoreType.DMA,
            pltpu.SemaphoreType.DMA,
        ],
    )
    return k2b, k3, k5


def kernel(x, W_router, W1, W2, noise, token_mask):
    k2b, k3, k5 = _sc_kernels()
    xf = x.reshape(N, D_MODEL)
    gate2, eidx2, lb11, z11 = _k1(xf, W_router, noise)
    src2, keep2 = _k2(gate2, eidx2)
    src1 = src2.reshape(N)
    topi, gslot = k2b(src1, keep2.reshape(N), gate2.reshape(N))
    xg = k3(topi, xf)
    ybuf = _k4(xg, W1, W2, gslot.reshape(NSLOT, 1))
    y = k5(src1, ybuf.reshape(NSLOT, D_MODEL))
    return (y.reshape(x.shape), lb11[0, 0], z11[0, 0])
